# baseline (speedup 1.0000x reference)
import functools

import jax
import jax.numpy as jnp
from jax.experimental import pallas as pl
from jax.experimental.pallas import tpu as pltpu
from jax.experimental.pallas import tpu_sc as plsc

_BS = 512


def _tc_body(start_ref, pe_ref, x_ref, o_ref):
    i = pl.program_id(0)
    row0 = pl.multiple_of(start_ref[0] + i * _BS, 8)
    pe_blk = pe_ref[pl.ds(row0, _BS), :]
    o_ref[...] = x_ref[...] + pe_blk[None, :, :]


@jax.jit
def _tc_pe_add(x, position_embedding, start):
    B, S, D = x.shape
    return pl.pallas_call(
        _tc_body,
        grid_spec=pltpu.PrefetchScalarGridSpec(
            num_scalar_prefetch=1,
            grid=(S // _BS,),
            in_specs=[
                pl.BlockSpec(position_embedding.shape, lambda i, s_ref: (0, 0)),
                pl.BlockSpec((B, _BS, D), lambda i, s_ref: (0, i, 0)),
            ],
            out_specs=pl.BlockSpec((B, _BS, D), lambda i, s_ref: (0, i, 0)),
        ),
        out_shape=jax.ShapeDtypeStruct(x.shape, x.dtype),
        compiler_params=pltpu.CompilerParams(
            dimension_semantics=("parallel",),
        ),
    )(start, position_embedding, x)


_SC_BLK = 16
_SC_LANES = 16


@jax.jit
def _sc_pe_add(x2d, position_embedding):
    N, D = x2d.shape
    S = position_embedding.shape[0]
    n_pe_blocks = S // _SC_BLK
    mesh = plsc.VectorSubcoreMesh(core_axis_name="c", subcore_axis_name="s")

    @functools.partial(
        pl.kernel,
        out_type=jax.ShapeDtypeStruct((N, D), x2d.dtype),
        mesh=mesh,
    )
    def sc_kernel(x_hbm, pe_hbm, o_hbm):
        def body(x_blk, pe_blk, o_blk):
            @pl.loop(0, _SC_BLK)
            def _(r):
                @pl.loop(0, D, step=_SC_LANES)
                def _(c):
                    slc = (pl.ds(r, 1), pl.ds(c, _SC_LANES))
                    o_blk.at[*slc][...] = (
                        x_blk.at[*slc][...] + pe_blk.at[*slc][...]
                    )

        pltpu.emit_pipeline(
            body,
            grid=(N // _SC_BLK,),
            in_specs=[
                pl.BlockSpec((_SC_BLK, D), lambda i: (i, 0)),
                pl.BlockSpec((_SC_BLK, D), lambda i: (i % n_pe_blocks, 0)),
            ],
            out_specs=[pl.BlockSpec((_SC_BLK, D), lambda i: (i, 0))],
            core_axis_name=("c", "s"),
            dimension_semantics=(pltpu.PARALLEL,),
        )(x_hbm, pe_hbm, o_hbm)

    return sc_kernel(x2d, position_embedding)


def kernel(x, position_embedding, position_start):
    B, S, D = x.shape
    out2d = _sc_pe_add(x.reshape(B * S, D), position_embedding)
    return out2d.reshape(B, S, D)


# speedup vs baseline: 4.6688x; 4.6688x over previous
import jax
import jax.numpy as jnp
from jax.experimental import pallas as pl
from jax.experimental.pallas import tpu as pltpu

_DB = 256


def _tc_body(start_ref, pe_ref, x_ref, o_ref):
    row0 = pl.multiple_of(start_ref[0], 8)
    S = x_ref.shape[1]
    pe_blk = pe_ref[pl.ds(row0, S), :]
    o_ref[...] = x_ref[...] + pe_blk[None, :, :]


@jax.jit
def _tc_pe_add(x, position_embedding, start):
    B, S, D = x.shape
    M = position_embedding.shape[0]
    return pl.pallas_call(
        _tc_body,
        grid_spec=pltpu.PrefetchScalarGridSpec(
            num_scalar_prefetch=1,
            grid=(D // _DB,),
            in_specs=[
                pl.BlockSpec((M, _DB), lambda i, s_ref: (0, i)),
                pl.BlockSpec((B, S, _DB), lambda i, s_ref: (0, 0, i)),
            ],
            out_specs=pl.BlockSpec((B, S, _DB), lambda i, s_ref: (0, 0, i)),
        ),
        out_shape=jax.ShapeDtypeStruct(x.shape, x.dtype),
        compiler_params=pltpu.CompilerParams(
            dimension_semantics=("parallel",),
        ),
    )(start, position_embedding, x)


def kernel(x, position_embedding, position_start):
    start = jnp.asarray(position_start, jnp.int32).reshape((1,))
    return _tc_pe_add(x, position_embedding, start)
